# baseline (speedup 1.0000x reference)
import functools

import jax
import jax.numpy as jnp
from jax import lax
from jax.experimental import pallas as pl
from jax.experimental.pallas import tpu as pltpu
from jax.experimental.pallas import tpu_sc as plsc

B = 16384
N = 128
NB = N + 1
J = 65
NO = J - 1
EPS = 1e-5
NEAR = 0.05
FAR = 6.0

NUM_CORES = 2
NUM_SUBCORES = 16
NW = NUM_CORES * NUM_SUBCORES
RAYS_PER_W = B // NW
C = 128
G = C // 16
CHUNKS = RAYS_PER_W // C

_mesh = plsc.VectorSubcoreMesh(core_axis_name="c", subcore_axis_name="s")

_f32 = jnp.float32
_i32 = jnp.int32


def _body(w_hbm, ebt_hbm,
          bs_hbm, be_hbm, ss_hbm, se_hbm,
          wbuf0, wbuf1, ebbuf0, ebbuf1, cdfbuf, mbuf,
          binsbuf0, binsbuf1, euclbuf0, euclbuf1,
          sem_w0, sem_w1, sem_e0, sem_e1, sem_o0, sem_o1):
    wbufs = [wbuf0, wbuf1]
    ebbufs = [ebbuf0, ebbuf1]
    binsbufs = [binsbuf0, binsbuf1]
    euclbufs = [euclbuf0, euclbuf1]
    sem_w = [sem_w0, sem_w1]
    sem_e = [sem_e0, sem_e1]
    sem_o = [sem_o0, sem_o1]

    wid = lax.axis_index("s") * NUM_CORES + lax.axis_index("c")
    lane = lax.broadcasted_iota(_i32, (16,), 0)
    zf = jnp.zeros((16,), _f32)
    zi = jnp.zeros((16,), _i32)
    ones_i = jnp.ones((16,), _i32)

    @plsc.parallel_loop(0, J + 1, unroll=6)
    def _minit(j):
        mbuf[j, :] = zi

    def start_in(ci):
        slot = ci % 2
        base = wid * RAYS_PER_W + ci * C
        hw = pltpu.async_copy(w_hbm.at[pl.ds(base * N, C * N)],
                              wbufs[slot], sem_w[slot])
        he = pltpu.async_copy(ebt_hbm.at[:, pl.ds(base, C)],
                              ebbufs[slot], sem_e[slot])
        return hw, he

    hin = {0: start_in(0), 1: start_in(1)}
    hout = {}
    for ci in range(CHUNKS):
        slot = ci % 2
        hw, he = hin[ci]
        hw.wait()
        he.wait()
        if ci >= 2:
            for h in hout[ci - 2]:
                h.wait()
        wbuf = wbufs[slot]
        ebbuf = ebbufs[slot]
        binsbuf = binsbufs[slot]
        euclbuf = euclbufs[slot]

        def group_body(g, _, wbuf=wbuf, ebbuf=ebbuf,
                       binsbuf=binsbuf, euclbuf=euclbuf):
            crow = g * 16 + lane
            wbase = crow * N

            cdfbuf[pl.ds(0, 16)] = zf

            @plsc.parallel_loop(0, N, step=8, unroll=2, carry=zf)
            def total(i, acc):
                ib = wbase + i
                w = [plsc.load_gather(wbuf, [ib + k]) for k in range(8)]
                s01 = w[0] + w[1]
                s23 = w[2] + w[3]
                s45 = w[4] + w[5]
                s67 = w[6] + w[7]
                s03 = s01 + s23
                s47 = s45 + s67
                p = [w[0], s01, s01 + w[2], s03, s03 + w[4], s03 + s45,
                     s03 + s45 + w[6], s03 + s47]
                for k in range(8):
                    cdfbuf[pl.ds((i + 1 + k) * 16, 16)] = acc + p[k]
                return acc + p[7]

            pad = jnp.maximum(EPS - total, 0.0)
            inv = 1.0 / (total + pad)
            padper = pad * (1.0 / N)

            @plsc.parallel_loop(0, N, unroll=16)
            def _pb(i):
                raw = cdfbuf[pl.ds((i + 1) * 16, 16)]
                fi = (i + 1).astype(_f32)
                c = jnp.minimum((raw + padper * fi) * inv, 1.0)
                cdfbuf[pl.ds((i + 1) * 16, 16)] = c
                k = (c * float(J) + 0.5).astype(_i32)
                plsc.addupdate_scatter(mbuf, [k, lane], ones_i)

            col0 = g * 16

            @plsc.parallel_loop(0, J, unroll=5, carry=zi)
            def _run(j, run):
                run = run + mbuf[j, :]
                mbuf[j, :] = zi
                below = run
                ic0 = (below << 4) + lane
                c0 = plsc.load_gather(cdfbuf, [ic0])
                c1 = plsc.load_gather(cdfbuf, [ic0 + 16])
                e0 = plsc.load_gather(ebbuf, [below, crow])
                e1 = plsc.load_gather(ebbuf, [below + 1, crow])
                uu = ((2 * j + 1).astype(_f32)) * _f32(1.0 / (2 * J))
                denom = c1 - c0
                denom = jnp.where(denom < 1e-5, 1.0, denom)
                t = jnp.clip((uu - c0) / denom, 0.0, 1.0)
                binsv = e0 + t * (e1 - e0)
                eucl = NEAR + binsv * (FAR - NEAR)
                binsbuf[j, pl.ds(col0, 16)] = binsv
                euclbuf[j, pl.ds(col0, 16)] = eucl
                return run

            mbuf[J, :] = zi
            return 0
        lax.fori_loop(0, G, group_body, 0)

        base = wid * RAYS_PER_W + ci * C
        s = sem_o[slot]
        hout[ci] = [
            pltpu.async_copy(euclbuf.at[pl.ds(0, NO)],
                             bs_hbm.at[:, pl.ds(base, C)], s),
            pltpu.async_copy(euclbuf.at[pl.ds(1, NO)],
                             be_hbm.at[:, pl.ds(base, C)], s),
            pltpu.async_copy(binsbuf.at[pl.ds(0, NO)],
                             ss_hbm.at[:, pl.ds(base, C)], s),
            pltpu.async_copy(binsbuf.at[pl.ds(1, NO)],
                             se_hbm.at[:, pl.ds(base, C)], s),
        ]
        if ci + 2 < CHUNKS:
            hin[ci + 2] = start_in(ci + 2)
    for ci in (CHUNKS - 2, CHUNKS - 1):
        for h in hout[ci]:
            h.wait()


_sampler = functools.partial(
    pl.kernel,
    mesh=_mesh,
    compiler_params=pltpu.CompilerParams(
        needs_layout_passes=False, use_tc_tiling_on_sc=False),
    out_type=[jax.ShapeDtypeStruct((NO, B), _f32)] * 4,
    scratch_types=[
        pltpu.VMEM((C * N,), _f32),
        pltpu.VMEM((C * N,), _f32),
        pltpu.VMEM((NB, C), _f32),
        pltpu.VMEM((NB, C), _f32),
        pltpu.VMEM((NB * 16,), _f32),
        pltpu.VMEM((J + 1, 16), _i32),
        pltpu.VMEM((J, C), _f32),
        pltpu.VMEM((J, C), _f32),
        pltpu.VMEM((J, C), _f32),
        pltpu.VMEM((J, C), _f32),
        pltpu.SemaphoreType.DMA,
        pltpu.SemaphoreType.DMA,
        pltpu.SemaphoreType.DMA,
        pltpu.SemaphoreType.DMA,
        pltpu.SemaphoreType.DMA,
        pltpu.SemaphoreType.DMA,
    ],
)(_body)


def kernel(weights, existing_bins):
    wf = weights.reshape(B * N)
    ebt = jnp.transpose(existing_bins)
    bs, be, ss, se = _sampler(wf, ebt)
    def _t(x):
        return jnp.transpose(x)[:, :, None]
    return (_t(bs), _t(be), _t(ss), _t(se))


# speedup vs baseline: 1.0086x; 1.0086x over previous
import functools

import jax
import jax.numpy as jnp
from jax import lax
from jax.experimental import pallas as pl
from jax.experimental.pallas import tpu as pltpu
from jax.experimental.pallas import tpu_sc as plsc

B = 16384
N = 128
NB = N + 1
J = 65
NO = J - 1
EPS = 1e-5
NEAR = 0.05
FAR = 6.0

NUM_CORES = 2
NUM_SUBCORES = 16
NW = NUM_CORES * NUM_SUBCORES
RAYS_PER_W = B // NW
C = 64
G = C // 16
CHUNKS = RAYS_PER_W // C

_mesh = plsc.VectorSubcoreMesh(core_axis_name="c", subcore_axis_name="s")

_f32 = jnp.float32
_i32 = jnp.int32


def _body(w_hbm, ebt_hbm,
          bs_hbm, be_hbm, ss_hbm, se_hbm,
          wbuf0, wbuf1, ebbuf0, ebbuf1, cdfbuf, mbuf,
          binsbuf0, binsbuf1, euclbuf0, euclbuf1,
          sem_w0, sem_w1, sem_e0, sem_e1, sem_o0, sem_o1):
    wbufs = [wbuf0, wbuf1]
    ebbufs = [ebbuf0, ebbuf1]
    binsbufs = [binsbuf0, binsbuf1]
    euclbufs = [euclbuf0, euclbuf1]
    sem_w = [sem_w0, sem_w1]
    sem_e = [sem_e0, sem_e1]
    sem_o = [sem_o0, sem_o1]

    wid = lax.axis_index("s") * NUM_CORES + lax.axis_index("c")
    lane = lax.broadcasted_iota(_i32, (16,), 0)
    zf = jnp.zeros((16,), _f32)
    zi = jnp.zeros((16,), _i32)
    ones_i = jnp.ones((16,), _i32)

    @plsc.parallel_loop(0, J + 1, unroll=6)
    def _minit(j):
        mbuf[j, :] = zi

    def start_in(ci):
        slot = ci % 2
        base = wid * RAYS_PER_W + ci * C
        hw = pltpu.async_copy(w_hbm.at[pl.ds(base * N, C * N)],
                              wbufs[slot], sem_w[slot])
        he = pltpu.async_copy(ebt_hbm.at[:, pl.ds(base, C)],
                              ebbufs[slot], sem_e[slot])
        return hw, he

    hin = {0: start_in(0), 1: start_in(1)}
    hout = {}
    for ci in range(CHUNKS):
        slot = ci % 2
        hw, he = hin[ci]
        hw.wait()
        he.wait()
        if ci >= 2:
            for h in hout[ci - 2]:
                h.wait()
        wbuf = wbufs[slot]
        ebbuf = ebbufs[slot]
        binsbuf = binsbufs[slot]
        euclbuf = euclbufs[slot]

        def group_body(g, _, wbuf=wbuf, ebbuf=ebbuf,
                       binsbuf=binsbuf, euclbuf=euclbuf):
            crow = g * 16 + lane
            wbase = crow * N

            cdfbuf[pl.ds(0, 16)] = zf

            @plsc.parallel_loop(0, N, step=8, unroll=2, carry=zf)
            def total(i, acc):
                ib = wbase + i
                w = [plsc.load_gather(wbuf, [ib + k]) for k in range(8)]
                s01 = w[0] + w[1]
                s23 = w[2] + w[3]
                s45 = w[4] + w[5]
                s67 = w[6] + w[7]
                s03 = s01 + s23
                s47 = s45 + s67
                p = [w[0], s01, s01 + w[2], s03, s03 + w[4], s03 + s45,
                     s03 + s45 + w[6], s03 + s47]
                for k in range(8):
                    cdfbuf[pl.ds((i + 1 + k) * 16, 16)] = acc + p[k]
                return acc + p[7]

            pad = jnp.maximum(EPS - total, 0.0)
            inv = 1.0 / (total + pad)
            padper = pad * (1.0 / N)

            @plsc.parallel_loop(0, N, unroll=8)
            def _pb(i):
                raw = cdfbuf[pl.ds((i + 1) * 16, 16)]
                fi = (i + 1).astype(_f32)
                c = jnp.minimum((raw + padper * fi) * inv, 1.0)
                cdfbuf[pl.ds((i + 1) * 16, 16)] = c
                k = (c * float(J) + 0.5).astype(_i32)
                plsc.addupdate_scatter(mbuf, [k, lane], ones_i)

            col0 = g * 16

            @plsc.parallel_loop(0, J, unroll=5, carry=zi)
            def _run(j, run):
                run = run + mbuf[j, :]
                mbuf[j, :] = zi
                below = run
                ic0 = (below << 4) + lane
                c0 = plsc.load_gather(cdfbuf, [ic0])
                c1 = plsc.load_gather(cdfbuf, [ic0 + 16])
                e0 = plsc.load_gather(ebbuf, [below, crow])
                e1 = plsc.load_gather(ebbuf, [below + 1, crow])
                uu = ((2 * j + 1).astype(_f32)) * _f32(1.0 / (2 * J))
                denom = c1 - c0
                denom = jnp.where(denom < 1e-5, 1.0, denom)
                t = jnp.clip((uu - c0) / denom, 0.0, 1.0)
                binsv = e0 + t * (e1 - e0)
                eucl = NEAR + binsv * (FAR - NEAR)
                binsbuf[j, pl.ds(col0, 16)] = binsv
                euclbuf[j, pl.ds(col0, 16)] = eucl
                return run

            mbuf[J, :] = zi
            return 0
        lax.fori_loop(0, G, group_body, 0)

        base = wid * RAYS_PER_W + ci * C
        s = sem_o[slot]
        hout[ci] = [
            pltpu.async_copy(euclbuf.at[pl.ds(0, NO)],
                             bs_hbm.at[:, pl.ds(base, C)], s),
            pltpu.async_copy(euclbuf.at[pl.ds(1, NO)],
                             be_hbm.at[:, pl.ds(base, C)], s),
            pltpu.async_copy(binsbuf.at[pl.ds(0, NO)],
                             ss_hbm.at[:, pl.ds(base, C)], s),
            pltpu.async_copy(binsbuf.at[pl.ds(1, NO)],
                             se_hbm.at[:, pl.ds(base, C)], s),
        ]
        if ci + 2 < CHUNKS:
            hin[ci + 2] = start_in(ci + 2)
    for ci in (CHUNKS - 2, CHUNKS - 1):
        for h in hout[ci]:
            h.wait()


_sampler = functools.partial(
    pl.kernel,
    mesh=_mesh,
    compiler_params=pltpu.CompilerParams(
        needs_layout_passes=False, use_tc_tiling_on_sc=False),
    out_type=[jax.ShapeDtypeStruct((NO, B), _f32)] * 4,
    scratch_types=[
        pltpu.VMEM((C * N,), _f32),
        pltpu.VMEM((C * N,), _f32),
        pltpu.VMEM((NB, C), _f32),
        pltpu.VMEM((NB, C), _f32),
        pltpu.VMEM((NB * 16,), _f32),
        pltpu.VMEM((J + 1, 16), _i32),
        pltpu.VMEM((J, C), _f32),
        pltpu.VMEM((J, C), _f32),
        pltpu.VMEM((J, C), _f32),
        pltpu.VMEM((J, C), _f32),
        pltpu.SemaphoreType.DMA,
        pltpu.SemaphoreType.DMA,
        pltpu.SemaphoreType.DMA,
        pltpu.SemaphoreType.DMA,
        pltpu.SemaphoreType.DMA,
        pltpu.SemaphoreType.DMA,
    ],
)(_body)


def kernel(weights, existing_bins):
    wf = weights.reshape(B * N)
    ebt = jnp.transpose(existing_bins)
    bs, be, ss, se = _sampler(wf, ebt)
    def _t(x):
        return jnp.transpose(x)[:, :, None]
    return (_t(bs), _t(be), _t(ss), _t(se))
